# Initial kernel scaffold; baseline (speedup 1.0000x reference)
import jax
import jax.numpy as jnp
from jax.experimental import pallas as pl


def kernel(x, enc_W1, enc_b1, enc_g1, enc_be1, enc_W2, enc_b2, enc_g2, enc_be2, enc_W3, enc_b3, codebook, dec_W1, dec_b1, dec_g1, dec_be1, dec_W2, dec_b2, dec_g2, dec_be2, dec_W3, dec_b3):
    raise NotImplementedError("write your pallas kernel here")


# speedup vs baseline: 1.2771x; 1.2771x over previous
import jax
import jax.numpy as jnp
from jax.experimental import pallas as pl
from jax.experimental.pallas import tpu as pltpu
from jax.experimental.pallas import tpu_sc as plsc

_B = 512
_BATCH = 4096
_D_IN = 1024
_D_H = 2048
_D_LAT = 256
_N_CODES = 16384
_CTILE = 2048
_CHUNK_BOUNDS = (0, 4096, 8192, 12288, 16384)
_GW = 128


def _ln(h, g, be):
    mu = jnp.mean(h, axis=-1, keepdims=True)
    var = jnp.mean((h - mu) ** 2, axis=-1, keepdims=True)
    return (h - mu) / jnp.sqrt(var + 1e-5) * g + be


def _mlp_core(xb, W1_ref, b1_ref, g1_ref, be1_ref, W2_ref, b2_ref, g2_ref,
              be2_ref, W3_ref, b3_ref):
    h = jnp.dot(xb, W1_ref[...], preferred_element_type=jnp.float32) + b1_ref[...]
    h = jax.nn.relu(_ln(h, g1_ref[...], be1_ref[...]))
    h = jnp.dot(h.astype(jnp.bfloat16), W2_ref[...],
                preferred_element_type=jnp.float32) + b2_ref[...]
    h = jax.nn.relu(_ln(h, g2_ref[...], be2_ref[...]))
    return jnp.dot(h.astype(jnp.bfloat16), W3_ref[...],
                   preferred_element_type=jnp.float32) + b3_ref[...]


def _dist_body(z_ref, zsq_ref, csq_ref, cb_ref, idx_ref, cbb):
    i = pl.program_id(0)

    @pl.when(i == 0)
    def _init():
        cbb[...] = cb_ref[...].astype(jnp.bfloat16)

    z_sq = zsq_ref[...]
    zb = z_ref[...].astype(jnp.bfloat16)
    nch = len(_CHUNK_BOUNDS) - 1
    rm = [jnp.full((_B, 1), 3.0e38, jnp.float32) for _ in range(nch)]
    ri = [jnp.zeros((_B, 1), jnp.float32) for _ in range(nch)]
    for t in range(_N_CODES // _CTILE):
        cbt = cbb[pl.ds(t * _CTILE, _CTILE), :]
        mm = jax.lax.dot_general(zb, cbt, (((1,), (1,)), ((), ())),
                                 preferred_element_type=jnp.float32)
        dist = (z_sq + csq_ref[:, pl.ds(t * _CTILE, _CTILE)]) - 2.0 * mm
        t0, t1 = t * _CTILE, (t + 1) * _CTILE
        for c in range(nch):
            lo = max(t0, _CHUNK_BOUNDS[c])
            hi = min(t1, _CHUNK_BOUNDS[c + 1])
            if lo >= hi:
                continue
            seg = dist[:, lo - t0:hi - t0]
            w = hi - lo
            pm = jnp.min(seg, axis=1, keepdims=True)
            cols = jax.lax.broadcasted_iota(jnp.int32, (_B, w), 1).astype(jnp.float32)
            pa = jnp.min(jnp.where(seg == pm, cols, jnp.float32(3.0e38)),
                         axis=1, keepdims=True) + jnp.float32(lo)
            upd = pm < rm[c]
            ri[c] = jnp.where(upd, pa, ri[c])
            rm[c] = jnp.where(upd, pm, rm[c])
    m = jnp.full((_B, 1), 3.0e38, jnp.float32)
    run_idx = jnp.zeros((_B, 1), jnp.float32)
    for c in range(nch):
        upd = rm[c] < m
        run_idx = jnp.where(upd, ri[c], run_idx)
        m = jnp.where(upd, rm[c].astype(jnp.bfloat16).astype(jnp.float32), m)
    idx_ref[...] = run_idx.astype(jnp.int32)


def _dist_argmin(z, z_sq, c_sq, codebook):
    return pl.pallas_call(
        _dist_body,
        grid=(_BATCH // _B,),
        in_specs=[
            pl.BlockSpec((_B, _D_LAT), lambda i: (i, 0)),
            pl.BlockSpec((_B, 1), lambda i: (i, 0)),
            pl.BlockSpec((1, _N_CODES), lambda i: (0, 0)),
            pl.BlockSpec((_N_CODES, _D_LAT), lambda i: (0, 0)),
        ],
        out_specs=pl.BlockSpec((_B, 1), lambda i: (i, 0)),
        out_shape=jax.ShapeDtypeStruct((_BATCH, 1), jnp.int32),
        scratch_shapes=[
            pltpu.VMEM((_N_CODES, _D_LAT), jnp.bfloat16),
        ],
    )(z, z_sq, c_sq, codebook)


def _sc_gather(codebook, idx):
    idx2 = idx.reshape(1, _BATCH)
    mesh = plsc.VectorSubcoreMesh(core_axis_name="c", subcore_axis_name="s")

    @pl.kernel(out_type=jax.ShapeDtypeStruct((_BATCH, _D_LAT), jnp.float32),
               mesh=mesh)
    def gk(cb_hbm, i_hbm, o_hbm):
        def body(i_vmem, o_vmem):
            pltpu.sync_copy(cb_hbm.at[i_vmem.at[0]], o_vmem)

        pltpu.emit_pipeline(
            body,
            grid=(_BATCH // _GW,),
            in_specs=[pl.BlockSpec((1, _GW), lambda i: (0, i))],
            out_specs=[pl.BlockSpec((_GW, _D_LAT), lambda i: (i, 0))],
            core_axis_name=("c", "s"),
            dimension_semantics=(pltpu.PARALLEL,),
        )(i_hbm, o_hbm)

    return gk(codebook, idx2)


def _dec_body(z_ref, zq_ref, x_ref, W1, b1, g1, be1, W2, b2, g2, be2, W3, b3,
              recon_ref, vq_ref, rs_ref):
    z32 = z_ref[...]
    zq = zq_ref[...]
    zin = z32 + (zq - z32)
    d = z32 - zq
    vq_ref[...] = jnp.sum(d * d, axis=(0, 1), keepdims=True)[None, :, :]
    recon = _mlp_core(zin.astype(jnp.bfloat16),
                      W1, b1, g1, be1, W2, b2, g2, be2, W3, b3)
    recon_ref[...] = recon
    r = recon - x_ref[...]
    rs_ref[...] = jnp.sum(r * r, axis=(0, 1), keepdims=True)[None, :, :]


def _decoder(z, z_q, x, W1, b1, g1, be1, W2, b2, g2, be2, W3, b3):
    const = lambda i: (0, 0)
    row = lambda i: (i, 0)
    vec = pl.BlockSpec((1, _D_H), const)
    scal = pl.BlockSpec((1, 1, 1), lambda i: (i, 0, 0))
    return pl.pallas_call(
        _dec_body,
        grid=(_BATCH // _B,),
        in_specs=[
            pl.BlockSpec((_B, _D_LAT), row),
            pl.BlockSpec((_B, _D_LAT), row),
            pl.BlockSpec((_B, _D_IN), row),
            pl.BlockSpec((_D_LAT, _D_H), const), vec, vec, vec,
            pl.BlockSpec((_D_H, _D_H), const), vec, vec, vec,
            pl.BlockSpec((_D_H, _D_IN), const),
            pl.BlockSpec((1, _D_IN), const),
        ],
        out_specs=[
            pl.BlockSpec((_B, _D_IN), row),
            scal, scal,
        ],
        out_shape=[
            jax.ShapeDtypeStruct((_BATCH, _D_IN), jnp.float32),
            jax.ShapeDtypeStruct((_BATCH // _B, 1, 1), jnp.float32),
            jax.ShapeDtypeStruct((_BATCH // _B, 1, 1), jnp.float32),
        ],
    )(z, z_q, x, W1, b1, g1, be1, W2, b2, g2, be2, W3, b3)


def kernel(x, enc_W1, enc_b1, enc_g1, enc_be1, enc_W2, enc_b2, enc_g2,
           enc_be2, enc_W3, enc_b3, codebook, dec_W1, dec_b1, dec_g1, dec_be1,
           dec_W2, dec_b2, dec_g2, dec_be2, dec_W3, dec_b3):
    bf = jnp.bfloat16
    r2 = lambda a: a.reshape(1, -1)

    h = jax.nn.relu(_ln(x @ enc_W1 + enc_b1, enc_g1, enc_be1))
    h = jax.nn.relu(_ln(h @ enc_W2 + enc_b2, enc_g2, enc_be2))
    z = h @ enc_W3 + enc_b3
    z_sq = jnp.sum(z ** 2, axis=1, keepdims=True)
    c_sq = jnp.sum(codebook ** 2, axis=1).reshape(1, _N_CODES)

    idx2 = _dist_argmin(z, z_sq, c_sq, codebook)
    idx = idx2.reshape(_BATCH)
    z_q = _sc_gather(codebook, idx)
    recon, vq_p, rs_p = _decoder(
        z, z_q, x, dec_W1.astype(bf), r2(dec_b1), r2(dec_g1), r2(dec_be1),
        dec_W2.astype(bf), r2(dec_b2), r2(dec_g2), r2(dec_be2),
        dec_W3.astype(bf), r2(dec_b3))

    vq_m = jnp.sum(vq_p) / (_BATCH * _D_LAT)
    recon_loss = jnp.sum(rs_p) / (_BATCH * _D_IN)
    vq_loss = vq_m + 0.25 * vq_m
    total_loss = recon_loss + vq_loss
    return (recon, idx, total_loss, recon_loss)


# speedup vs baseline: 1.2864x; 1.0073x over previous
import jax
import jax.numpy as jnp
from jax.experimental import pallas as pl
from jax.experimental.pallas import tpu as pltpu
from jax.experimental.pallas import tpu_sc as plsc

_B = 512
_BATCH = 4096
_D_IN = 1024
_D_H = 2048
_D_LAT = 256
_N_CODES = 16384
_CTILE = 2048
_CHUNK_BOUNDS = (0, 4096, 8192, 12288, 16384)
_GW = 128


def _ln(h, g, be):
    mu = jnp.mean(h, axis=-1, keepdims=True)
    var = jnp.mean((h - mu) ** 2, axis=-1, keepdims=True)
    return (h - mu) / jnp.sqrt(var + 1e-5) * g + be


def _mlp_core(xb, W1_ref, b1_ref, g1_ref, be1_ref, W2_ref, b2_ref, g2_ref,
              be2_ref, W3_ref, b3_ref):
    h = jnp.dot(xb, W1_ref[...], preferred_element_type=jnp.float32) + b1_ref[...]
    h = jax.nn.relu(_ln(h, g1_ref[...], be1_ref[...]))
    h = jnp.dot(h.astype(jnp.bfloat16), W2_ref[...],
                preferred_element_type=jnp.float32) + b2_ref[...]
    h = jax.nn.relu(_ln(h, g2_ref[...], be2_ref[...]))
    return jnp.dot(h.astype(jnp.bfloat16), W3_ref[...],
                   preferred_element_type=jnp.float32) + b3_ref[...]


def _dist_body(z_ref, zsq_ref, csq_ref, cb_ref, idx_ref):
    z_sq = zsq_ref[...]
    zb = z_ref[...].astype(jnp.bfloat16)
    nch = len(_CHUNK_BOUNDS) - 1
    rm = [None] * nch
    ri = [None] * nch
    for c in range(nch):
        lo, hi = _CHUNK_BOUNDS[c], _CHUNK_BOUNDS[c + 1]
        w = hi - lo
        cbt = cb_ref[pl.ds(lo, w), :]
        mm = jax.lax.dot_general(zb, cbt, (((1,), (1,)), ((), ())),
                                 preferred_element_type=jnp.float32)
        seg = (z_sq + csq_ref[:, pl.ds(lo, w)]) - 2.0 * mm
        pm = jnp.min(seg, axis=1, keepdims=True)
        cols = jax.lax.broadcasted_iota(jnp.int32, (_B, w), 1).astype(jnp.float32)
        pa = jnp.min(jnp.where(seg == pm, cols, jnp.float32(3.0e38)),
                     axis=1, keepdims=True) + jnp.float32(lo)
        rm[c] = pm
        ri[c] = pa
    m = jnp.full((_B, 1), 3.0e38, jnp.float32)
    run_idx = jnp.zeros((_B, 1), jnp.float32)
    for c in range(nch):
        upd = rm[c] < m
        run_idx = jnp.where(upd, ri[c], run_idx)
        m = jnp.where(upd, rm[c].astype(jnp.bfloat16).astype(jnp.float32), m)
    idx_ref[...] = run_idx.astype(jnp.int32)


def _dist_argmin(z, z_sq, c_sq, codebook_bf16):
    return pl.pallas_call(
        _dist_body,
        grid=(_BATCH // _B,),
        in_specs=[
            pl.BlockSpec((_B, _D_LAT), lambda i: (i, 0)),
            pl.BlockSpec((_B, 1), lambda i: (i, 0)),
            pl.BlockSpec((1, _N_CODES), lambda i: (0, 0)),
            pl.BlockSpec((_N_CODES, _D_LAT), lambda i: (0, 0)),
        ],
        out_specs=pl.BlockSpec((_B, 1), lambda i: (i, 0)),
        out_shape=jax.ShapeDtypeStruct((_BATCH, 1), jnp.int32),
    )(z, z_sq, c_sq, codebook_bf16)


def _sc_gather(codebook, idx):
    idx2 = idx.reshape(1, _BATCH)
    mesh = plsc.VectorSubcoreMesh(core_axis_name="c", subcore_axis_name="s")

    @pl.kernel(out_type=jax.ShapeDtypeStruct((_BATCH, _D_LAT), jnp.float32),
               mesh=mesh)
    def gk(cb_hbm, i_hbm, o_hbm):
        def body(i_vmem, o_vmem):
            pltpu.sync_copy(cb_hbm.at[i_vmem.at[0]], o_vmem)

        pltpu.emit_pipeline(
            body,
            grid=(_BATCH // _GW,),
            in_specs=[pl.BlockSpec((1, _GW), lambda i: (0, i))],
            out_specs=[pl.BlockSpec((_GW, _D_LAT), lambda i: (i, 0))],
            core_axis_name=("c", "s"),
            dimension_semantics=(pltpu.PARALLEL,),
        )(i_hbm, o_hbm)

    return gk(codebook, idx2)


def _dec_body(z_ref, zq_ref, x_ref, W1, b1, g1, be1, W2, b2, g2, be2, W3, b3,
              recon_ref, vq_ref, rs_ref):
    z32 = z_ref[...]
    zq = zq_ref[...]
    zin = z32 + (zq - z32)
    d = z32 - zq
    vq_ref[...] = jnp.sum(d * d, axis=(0, 1), keepdims=True)[None, :, :]
    recon = _mlp_core(zin.astype(jnp.bfloat16),
                      W1, b1, g1, be1, W2, b2, g2, be2, W3, b3)
    recon_ref[...] = recon
    r = recon - x_ref[...]
    rs_ref[...] = jnp.sum(r * r, axis=(0, 1), keepdims=True)[None, :, :]


def _decoder(z, z_q, x, W1, b1, g1, be1, W2, b2, g2, be2, W3, b3):
    const = lambda i: (0, 0)
    row = lambda i: (i, 0)
    vec = pl.BlockSpec((1, _D_H), const)
    scal = pl.BlockSpec((1, 1, 1), lambda i: (i, 0, 0))
    return pl.pallas_call(
        _dec_body,
        grid=(_BATCH // _B,),
        in_specs=[
            pl.BlockSpec((_B, _D_LAT), row),
            pl.BlockSpec((_B, _D_LAT), row),
            pl.BlockSpec((_B, _D_IN), row),
            pl.BlockSpec((_D_LAT, _D_H), const), vec, vec, vec,
            pl.BlockSpec((_D_H, _D_H), const), vec, vec, vec,
            pl.BlockSpec((_D_H, _D_IN), const),
            pl.BlockSpec((1, _D_IN), const),
        ],
        out_specs=[
            pl.BlockSpec((_B, _D_IN), row),
            scal, scal,
        ],
        out_shape=[
            jax.ShapeDtypeStruct((_BATCH, _D_IN), jnp.float32),
            jax.ShapeDtypeStruct((_BATCH // _B, 1, 1), jnp.float32),
            jax.ShapeDtypeStruct((_BATCH // _B, 1, 1), jnp.float32),
        ],
    )(z, z_q, x, W1, b1, g1, be1, W2, b2, g2, be2, W3, b3)


def kernel(x, enc_W1, enc_b1, enc_g1, enc_be1, enc_W2, enc_b2, enc_g2,
           enc_be2, enc_W3, enc_b3, codebook, dec_W1, dec_b1, dec_g1, dec_be1,
           dec_W2, dec_b2, dec_g2, dec_be2, dec_W3, dec_b3):
    bf = jnp.bfloat16
    r2 = lambda a: a.reshape(1, -1)

    h = jax.nn.relu(_ln(x @ enc_W1 + enc_b1, enc_g1, enc_be1))
    h = jax.nn.relu(_ln(h @ enc_W2 + enc_b2, enc_g2, enc_be2))
    z = h @ enc_W3 + enc_b3
    z_sq = jnp.sum(z ** 2, axis=1, keepdims=True)
    c_sq = jnp.sum(codebook ** 2, axis=1).reshape(1, _N_CODES)

    idx2 = _dist_argmin(z, z_sq, c_sq, codebook.astype(bf))
    idx = idx2.reshape(_BATCH)
    z_q = _sc_gather(codebook, idx)
    recon, vq_p, rs_p = _decoder(
        z, z_q, x, dec_W1.astype(bf), r2(dec_b1), r2(dec_g1), r2(dec_be1),
        dec_W2.astype(bf), r2(dec_b2), r2(dec_g2), r2(dec_be2),
        dec_W3.astype(bf), r2(dec_b3))

    vq_m = jnp.sum(vq_p) / (_BATCH * _D_LAT)
    recon_loss = jnp.sum(rs_p) / (_BATCH * _D_IN)
    vq_loss = vq_m + 0.25 * vq_m
    total_loss = recon_loss + vq_loss
    return (recon, idx, total_loss, recon_loss)
